# initial kernel scaffold (unmeasured)
import jax
import jax.numpy as jnp
from jax import lax
from jax.experimental import pallas as pl
from jax.experimental.pallas import tpu as pltpu

N_DEV = 4
E_PER_SHARD = 4
N_TOK = 4096
D = 1024
H = 2048
CAP = 1280
BM = 256


def _compute_body(e_ref, x_ref, w_ref, out_ref):
    acc = jnp.zeros((BM, H), jnp.float32)
    for j in range(E_PER_SHARD):
        m = e_ref[:, :] == j
        xm = jnp.where(m, x_ref[:, :], 0.0)
        acc = acc + jnp.dot(xm, w_ref[j], preferred_element_type=jnp.float32)
    out_ref[:, :] = acc


def _compute_slab(e_loc, x_loc, expert_W):
    return pl.pallas_call(
        _compute_body,
        grid=(CAP // BM,),
        in_specs=[
            pl.BlockSpec((BM, 1), lambda c: (c, 0)),
            pl.BlockSpec((BM, D), lambda c: (c, 0)),
            pl.BlockSpec((E_PER_SHARD, D, H), lambda c: (0, 0, 0)),
        ],
        out_specs=pl.BlockSpec((BM, H), lambda c: (c, 0)),
        out_shape=jax.ShapeDtypeStruct((CAP, H), jnp.float32),
    )(e_loc, x_loc, expert_W)


def _comm_body(slab_ref, out_ref, send_sems, recv_sems):
    my = lax.axis_index("i")
    out_ref[my] = slab_ref[:, :]
    copies = []
    for k in range(1, N_DEV):
        tgt = lax.rem(my + k, N_DEV)
        rdma = pltpu.make_async_remote_copy(
            src_ref=slab_ref,
            dst_ref=out_ref.at[my],
            send_sem=send_sems.at[k - 1],
            recv_sem=recv_sems.at[k - 1],
            device_id=(tgt,),
            device_id_type=pl.DeviceIdType.MESH,
        )
        rdma.start()
        copies.append(rdma)
    for r in copies:
        r.wait_send()
    for r in copies:
        r.wait_recv()


def _all_gather_slabs(slab):
    return pl.pallas_call(
        _comm_body,
        out_shape=jax.ShapeDtypeStruct((N_DEV, CAP, H), jnp.float32),
        in_specs=[pl.BlockSpec(memory_space=pltpu.VMEM)],
        out_specs=pl.BlockSpec(memory_space=pltpu.VMEM),
        scratch_shapes=[
            pltpu.SemaphoreType.DMA((N_DEV - 1,)),
            pltpu.SemaphoreType.DMA((N_DEV - 1,)),
        ],
    )(slab)


def kernel(x, router_W, route_idx, expert_W):
    del router_W
    e = route_idx[:, 0].astype(jnp.int32)

    order = jnp.argsort(e)
    sorted_e = e[order]
    x_sorted = x[order]
    shard_offsets = jnp.searchsorted(
        sorted_e, jnp.arange(N_DEV + 1, dtype=jnp.int32) * E_PER_SHARD
    ).astype(jnp.int32)

    my = lax.axis_index("i")
    start = shard_offsets[my]

    x_pad = jnp.concatenate([x_sorted, jnp.zeros((CAP, D), x.dtype)], axis=0)
    e_pad = jnp.concatenate(
        [sorted_e, jnp.full((CAP,), 10**6, jnp.int32)], axis=0
    )
    x_loc = lax.dynamic_slice(x_pad, (start, 0), (CAP, D))
    e_loc = (
        lax.dynamic_slice(e_pad, (start,), (CAP,)) - my * E_PER_SHARD
    ).reshape(CAP, 1)

    slab = _compute_slab(e_loc, x_loc, expert_W)
    slabs = _all_gather_slabs(slab)

    inv = jnp.argsort(order)
    owner = e // E_PER_SHARD
    r = inv - shard_offsets[owner]
    out = slabs.reshape(N_DEV * CAP, H)[owner * CAP + r]
    return out


# baseline (device time: 1127582 ns/iter reference)
import jax
import jax.numpy as jnp
from jax import lax
from jax.experimental import pallas as pl
from jax.experimental.pallas import tpu as pltpu

N_DEV = 4
E_PER_SHARD = 4
N_TOK = 4096
D = 1024
H = 2048
CAP = 1280
BM = 256


def _compute_body(e_ref, x_ref, w_ref, out_ref):
    j = pl.program_id(1)
    m = e_ref[:, :] == j
    xm = jnp.where(m, x_ref[:, :], 0.0)
    contrib = jnp.dot(xm, w_ref[0], preferred_element_type=jnp.float32)

    @pl.when(j == 0)
    def _():
        out_ref[:, :] = contrib

    @pl.when(j != 0)
    def _():
        out_ref[:, :] += contrib


def _compute_slab(e_loc, x_loc, expert_W):
    return pl.pallas_call(
        _compute_body,
        grid=(CAP // BM, E_PER_SHARD),
        in_specs=[
            pl.BlockSpec((BM, 1), lambda c, j: (c, 0)),
            pl.BlockSpec((BM, D), lambda c, j: (c, 0)),
            pl.BlockSpec((1, D, H), lambda c, j: (j, 0, 0)),
        ],
        out_specs=pl.BlockSpec((BM, H), lambda c, j: (c, 0)),
        out_shape=jax.ShapeDtypeStruct((CAP, H), jnp.float32),
    )(e_loc, x_loc, expert_W)


def _comm_body(slab_ref, out_ref, send_sems, recv_sems, local_sem):
    my = lax.axis_index("i")
    own = pltpu.make_async_copy(slab_ref, out_ref.at[my], local_sem)
    own.start()
    copies = []
    for k in range(1, N_DEV):
        tgt = lax.rem(my + k, N_DEV)
        rdma = pltpu.make_async_remote_copy(
            src_ref=slab_ref,
            dst_ref=out_ref.at[my],
            send_sem=send_sems.at[k - 1],
            recv_sem=recv_sems.at[k - 1],
            device_id=(tgt,),
            device_id_type=pl.DeviceIdType.MESH,
        )
        rdma.start()
        copies.append(rdma)
    own.wait()
    for r in copies:
        r.wait_send()
    for r in copies:
        r.wait_recv()


def _all_gather_slabs(slab):
    return pl.pallas_call(
        _comm_body,
        out_shape=jax.ShapeDtypeStruct((N_DEV, CAP, H), jnp.float32),
        in_specs=[pl.BlockSpec(memory_space=pltpu.VMEM)],
        out_specs=pl.BlockSpec(memory_space=pltpu.MemorySpace.HBM),
        scratch_shapes=[
            pltpu.SemaphoreType.DMA((N_DEV - 1,)),
            pltpu.SemaphoreType.DMA((N_DEV - 1,)),
            pltpu.SemaphoreType.DMA,
        ],
    )(slab)


def kernel(x, router_W, route_idx, expert_W):
    del router_W
    e = route_idx[:, 0].astype(jnp.int32)

    order = jnp.argsort(e)
    sorted_e = e[order]
    x_sorted = x[order]
    shard_offsets = jnp.searchsorted(
        sorted_e, jnp.arange(N_DEV + 1, dtype=jnp.int32) * E_PER_SHARD
    ).astype(jnp.int32)

    my = lax.axis_index("i")
    start = shard_offsets[my]

    x_pad = jnp.concatenate([x_sorted, jnp.zeros((CAP, D), x.dtype)], axis=0)
    e_pad = jnp.concatenate(
        [sorted_e, jnp.full((CAP,), 10**6, jnp.int32)], axis=0
    )
    x_loc = lax.dynamic_slice(x_pad, (start, 0), (CAP, D))
    e_loc = (
        lax.dynamic_slice(e_pad, (start,), (CAP,)) - my * E_PER_SHARD
    ).reshape(CAP, 1)

    slab = _compute_slab(e_loc, x_loc, expert_W)
    slabs = _all_gather_slabs(slab)

    inv = jnp.argsort(order)
    owner = e // E_PER_SHARD
    r = inv - shard_offsets[owner]
    out = slabs.reshape(N_DEV * CAP, H)[owner * CAP + r]
    return out
